# baseline (speedup 1.0000x reference)
import functools

import jax
import jax.numpy as jnp
from jax import lax
from jax.experimental import pallas as pl
from jax.experimental.pallas import tpu as pltpu
from jax.experimental.pallas import tpu_sc as plsc

B = 4096
CTX = 20
D = 128
V = 100000

NC = 2
NS = 16
NW = NC * NS
BPW = B // NW
CH = 4
NCHUNK = BPW // CH
IPC = CH * CTX


def _gather_mean_sc(idx_flat, emb):
  mesh = plsc.VectorSubcoreMesh(core_axis_name="c", subcore_axis_name="s")

  @functools.partial(
      pl.kernel,
      mesh=mesh,
      out_type=jax.ShapeDtypeStruct((B, D), jnp.float32),
      scratch_types=[
          pltpu.VMEM((BPW * CTX,), jnp.int32),
          pltpu.VMEM((2, IPC, D), jnp.float32),
          pltpu.VMEM((BPW, D), jnp.float32),
          pltpu.SemaphoreType.DMA,
          pltpu.SemaphoreType.DMA,
      ],
      compiler_params=pltpu.CompilerParams(use_tc_tiling_on_sc=True),
  )
  def k(idx_hbm, emb_hbm, ctx_hbm, idx_v, rows_v, acc_v, sem_a, sem_b):
    wid = lax.axis_index("s") * NC + lax.axis_index("c")
    base = wid * BPW
    pltpu.sync_copy(idx_hbm.at[pl.ds(base * CTX, BPW * CTX)], idx_v)
    pltpu.async_copy(
        emb_hbm.at[idx_v.at[pl.ds(0, IPC)]], rows_v.at[0], sem_a)

    def accumulate(buf, k_chunk):
      for r in range(CH):
        src = r * CTX
        for dd in range(D // 16):
          sl = pl.ds(dd * 16, 16)
          a = rows_v[buf, src, sl]
          for ci in range(1, CTX):
            a = a + rows_v[buf, src + ci, sl]
          acc_v[k_chunk * CH + r, sl] = a * (1.0 / CTX)

    def body(i, _):
      k0 = 2 * i
      k1 = k0 + 1
      pltpu.async_copy(
          emb_hbm.at[idx_v.at[pl.ds(k1 * IPC, IPC)]], rows_v.at[1], sem_b)
      pltpu.make_async_copy(
          emb_hbm.at[idx_v.at[pl.ds(k0 * IPC, IPC)]], rows_v.at[0],
          sem_a).wait()
      accumulate(0, k0)

      @pl.when(k0 + 2 < NCHUNK)
      def _():
        pltpu.async_copy(
            emb_hbm.at[idx_v.at[pl.ds((k0 + 2) * IPC, IPC)]], rows_v.at[0],
            sem_a)

      pltpu.make_async_copy(
          emb_hbm.at[idx_v.at[pl.ds(k1 * IPC, IPC)]], rows_v.at[1],
          sem_b).wait()
      accumulate(1, k1)
      return 0

    lax.fori_loop(0, NCHUNK // 2, body, 0)
    pltpu.sync_copy(acc_v, ctx_hbm.at[pl.ds(base, BPW)])

  return k(idx_flat, emb)


BN = 1792
GRID_N = pl.cdiv(V, BN)


def _proj_kernel(w_ref, ctx_ref, b_ref, out_ref):
  c = ctx_ref[...].astype(jnp.bfloat16)
  w = w_ref[...].astype(jnp.bfloat16)
  acc = lax.dot_general(w, c, (((1,), (1,)), ((), ())),
                        preferred_element_type=jnp.float32)
  b_col = jnp.transpose(b_ref[...], (1, 0))[:, 0:1]
  out_ref[...] = acc + b_col


def _project_t(ctx, W, b_col):
  return pl.pallas_call(
      _proj_kernel,
      grid=(GRID_N,),
      in_specs=[
          pl.BlockSpec((BN, D), lambda n: (n, 0)),
          pl.BlockSpec((B, D), lambda n: (0, 0)),
          pl.BlockSpec((8, BN), lambda n: (0, n)),
      ],
      out_specs=pl.BlockSpec((BN, B), lambda n: (n, 0)),
      out_shape=jax.ShapeDtypeStruct((V, B), jnp.float32),
      compiler_params=pltpu.CompilerParams(
          dimension_semantics=("parallel",),
          vmem_limit_bytes=63 * 1024 * 1024),
  )(W, ctx, b_col)


def kernel(X, emb, W, b):
  idx = X.astype(jnp.int32).reshape(-1)
  ctx = _gather_mean_sc(idx, emb)
  b8 = jnp.broadcast_to(b.reshape(1, V), (8, V))
  return _project_t(ctx, W, b8).T


# speedup vs baseline: 1.0655x; 1.0655x over previous
import functools

import jax
import jax.numpy as jnp
from jax import lax
from jax.experimental import pallas as pl
from jax.experimental.pallas import tpu as pltpu
from jax.experimental.pallas import tpu_sc as plsc

B = 4096
CTX = 20
D = 128
V = 100000

NC = 2
NS = 16
NW = NC * NS
BPW = B // NW
CH = 4
NCHUNK = BPW // CH
IPC = CH * CTX


def _gather_mean_sc(idx_flat, emb):
  mesh = plsc.VectorSubcoreMesh(core_axis_name="c", subcore_axis_name="s")

  @functools.partial(
      pl.kernel,
      mesh=mesh,
      out_type=jax.ShapeDtypeStruct((B, D), jnp.float32),
      scratch_types=[
          pltpu.VMEM((BPW * CTX,), jnp.int32),
          pltpu.VMEM((2, IPC, D), jnp.float32),
          pltpu.VMEM((BPW, D), jnp.float32),
          pltpu.SemaphoreType.DMA,
          pltpu.SemaphoreType.DMA,
      ],
      compiler_params=pltpu.CompilerParams(use_tc_tiling_on_sc=True),
  )
  def k(idx_hbm, emb_hbm, ctx_hbm, idx_v, rows_v, acc_v, sem_a, sem_b):
    wid = lax.axis_index("s") * NC + lax.axis_index("c")
    base = wid * BPW
    pltpu.sync_copy(idx_hbm.at[pl.ds(base * CTX, BPW * CTX)], idx_v)
    pltpu.async_copy(
        emb_hbm.at[idx_v.at[pl.ds(0, IPC)]], rows_v.at[0], sem_a)

    def accumulate(buf, k_chunk):
      def per_row(r, _):
        src = r * CTX
        for dd in range(D // 16):
          sl = pl.ds(dd * 16, 16)
          a = rows_v[buf, src, sl]
          for ci in range(1, CTX):
            a = a + rows_v[buf, src + ci, sl]
          acc_v[k_chunk * CH + r, sl] = a * (1.0 / CTX)
        return 0
      lax.fori_loop(0, CH, per_row, 0)

    def body(i, _):
      k0 = 2 * i
      k1 = k0 + 1
      pltpu.async_copy(
          emb_hbm.at[idx_v.at[pl.ds(k1 * IPC, IPC)]], rows_v.at[1], sem_b)
      pltpu.make_async_copy(
          emb_hbm.at[idx_v.at[pl.ds(k0 * IPC, IPC)]], rows_v.at[0],
          sem_a).wait()
      accumulate(0, k0)

      @pl.when(k0 + 2 < NCHUNK)
      def _():
        pltpu.async_copy(
            emb_hbm.at[idx_v.at[pl.ds((k0 + 2) * IPC, IPC)]], rows_v.at[0],
            sem_a)

      pltpu.make_async_copy(
          emb_hbm.at[idx_v.at[pl.ds(k1 * IPC, IPC)]], rows_v.at[1],
          sem_b).wait()
      accumulate(1, k1)
      return 0

    lax.fori_loop(0, NCHUNK // 2, body, 0)
    pltpu.sync_copy(acc_v, ctx_hbm.at[pl.ds(base, BPW)])

  return k(idx_flat, emb)


BN = 1792
GRID_N = pl.cdiv(V, BN)


def _proj_kernel(w_ref, ctx_ref, b_ref, out_ref):
  c = ctx_ref[...].astype(jnp.bfloat16)
  w = w_ref[...].astype(jnp.bfloat16)
  acc = lax.dot_general(w, c, (((1,), (1,)), ((), ())),
                        preferred_element_type=jnp.float32)
  b_col = jnp.transpose(b_ref[...], (1, 0))[:, 0:1]
  out_ref[...] = acc + b_col


def _project_t(ctx, W, b_col):
  return pl.pallas_call(
      _proj_kernel,
      grid=(GRID_N,),
      in_specs=[
          pl.BlockSpec((BN, D), lambda n: (n, 0)),
          pl.BlockSpec((B, D), lambda n: (0, 0)),
          pl.BlockSpec((8, BN), lambda n: (0, n)),
      ],
      out_specs=pl.BlockSpec((BN, B), lambda n: (n, 0)),
      out_shape=jax.ShapeDtypeStruct((V, B), jnp.float32),
      compiler_params=pltpu.CompilerParams(
          dimension_semantics=("parallel",),
          vmem_limit_bytes=63 * 1024 * 1024),
  )(W, ctx, b_col)


def kernel(X, emb, W, b):
  idx = X.astype(jnp.int32).reshape(-1)
  ctx = _gather_mean_sc(idx, emb)
  b8 = jnp.broadcast_to(b.reshape(1, V), (8, V))
  return _project_t(ctx, W, b8).T
